# Initial kernel scaffold; baseline (speedup 1.0000x reference)
import jax
import jax.numpy as jnp
from jax.experimental import pallas as pl


def kernel(logits, temperature, top_k, top_p, min_p, q_uniform):
    raise NotImplementedError("write your pallas kernel here")


# speedup vs baseline: 9.9746x; 9.9746x over previous
import functools

import jax
import jax.numpy as jnp
from jax import lax
from jax.experimental import pallas as pl
from jax.experimental.pallas import tpu as pltpu
from jax.experimental.pallas import tpu_sc as plsc

_EPS = 1e-05
B, V = 64, 100000
L = 16
NVREG = V // L
CH = 10000
NCH = V // CH
CHV = CH // L
NROWS_PER_W = 2
BIG_I32 = 2**30


def _iota16():
    return lax.broadcasted_iota(jnp.int32, (L,), 0)


def _splat_f(x):
    return jnp.full((L,), x, jnp.float32)


def _splat_i(x):
    return jnp.full((L,), x, jnp.int32)


def _row_scalar(vref, r):
    return vref[pl.ds(r, L)][0]


def _descend(cnt_ref, sum_ref, nbuckets, pos_k, tz1, s_base):
    ngroups = nbuckets // L
    it = _iota16()

    def body(g, carry):
        crun, brun, cbel, srun, selrun, sbel, lne = carry
        c16 = cnt_ref[pl.ds(g * L, L)]
        s16 = sum_ref[pl.ds(g * L, L)]
        cinc = plsc.cumsum(c16) + crun
        sinc = plsc.cumsum(s16) + srun
        mle = cinc <= pos_k
        msl = sinc <= tz1
        brun = brun + plsc.all_reduce_population_count(mle)
        selrun = selrun + plsc.all_reduce_population_count(msl)
        cbel = jnp.maximum(cbel, jnp.where(mle, cinc, 0))
        sbel = jnp.maximum(sbel, jnp.where(msl, sinc, s_base))
        gidx = _splat_i(g * L) + it
        lne = jnp.maximum(lne, jnp.where(s16 > 0.0, gidx, -1))
        crun = jnp.max(cinc)
        srun = jnp.max(sinc)
        return crun, brun, cbel, srun, selrun, sbel, lne

    init = (jnp.int32(0), _splat_i(0), _splat_i(0), jnp.float32(s_base),
            _splat_i(0), _splat_f(s_base), _splat_i(-1))
    _, brun, cbel, _, selrun, sbel, lne = lax.fori_loop(
        0, ngroups, body, init)
    return (jnp.max(brun), jnp.max(cbel), jnp.max(selrun), jnp.max(sbel),
            jnp.max(lne))


def _zero_hists(cnt_ref, sum_ref):
    zi = jnp.zeros((L,), jnp.int32)
    zf = jnp.zeros((L,), jnp.float32)

    def zbody(z, _):
        cnt_ref[pl.ds(z * L, L)] = zi
        sum_ref[pl.ds(z * L, L)] = zf
        return 0

    lax.fori_loop(0, 2048 // L, zbody, 0)


def _sc_sampler(logits, temperature, top_k, top_p, min_p, qrecip):
    mesh = plsc.VectorSubcoreMesh(core_axis_name="c", subcore_axis_name="s")

    @functools.partial(
        pl.kernel,
        out_type=jax.ShapeDtypeStruct((B * L,), jnp.int32),
        mesh=mesh,
        compiler_params=pltpu.CompilerParams(needs_layout_passes=False),
        scratch_types=[
            pltpu.VMEM((V,), jnp.float32),
            pltpu.VMEM((CH,), jnp.float32),
            pltpu.VMEM((CH,), jnp.float32),
            pltpu.VMEM((2048,), jnp.int32),
            pltpu.VMEM((2048,), jnp.float32),
            pltpu.VMEM((B + L,), jnp.float32),
            pltpu.VMEM((B + L,), jnp.int32),
            pltpu.VMEM((B + L,), jnp.float32),
            pltpu.VMEM((B + L,), jnp.float32),
            pltpu.VMEM((L,), jnp.int32),
            pltpu.SemaphoreType.DMA,
            pltpu.SemaphoreType.DMA,
        ],
    )
    def k(logits_hbm, temp_hbm, topk_hbm, topp_hbm, minp_hbm, qr_hbm,
          out_hbm, row_v, qr0_v, qr1_v, cnt_v, sum_v, t_v, k_v, p_v,
          mp_v, res_v, sem0, sem1):
        qrbufs = (qr0_v, qr1_v)
        wid = lax.axis_index("s") * 2 + lax.axis_index("c")
        pltpu.sync_copy(temp_hbm, t_v.at[pl.ds(0, B)])
        pltpu.sync_copy(topk_hbm, k_v.at[pl.ds(0, B)])
        pltpu.sync_copy(topp_hbm, p_v.at[pl.ds(0, B)])
        pltpu.sync_copy(minp_hbm, mp_v.at[pl.ds(0, B)])
        it = _iota16()
        sems = (sem0, sem1)

        for j in range(NROWS_PER_W):
            r = wid * NROWS_PER_W + j
            pltpu.sync_copy(logits_hbm.at[pl.ds(r * V, V)], row_v)
            t_sc = _row_scalar(t_v, r)
            k_sc = _row_scalar(k_v, r)
            p_sc = _row_scalar(p_v, r)
            mp_sc = _row_scalar(mp_v, r)
            t16 = _splat_f(t_sc)
            t_eff = jnp.where(t16 < _EPS, _splat_f(1.0), t16)
            rinv = _splat_f(1.0) / t_eff

            def body_a(i, carry):
                bmax, bidx = carry
                v = row_v[pl.ds(i * L, L)]
                gidx = _splat_i(i * L) + it
                take = v > bmax
                return jnp.where(take, v, bmax), jnp.where(take, gidx, bidx)

            bmax, bidx = lax.fori_loop(
                0, NVREG, body_a,
                (_splat_f(-jnp.inf), _splat_i(0)))
            m_raw = jnp.max(bmax)
            greedy = jnp.min(jnp.where(bmax == m_raw, bidx, BIG_I32))

            mx = _splat_f(m_raw) * rinv

            _zero_hists(cnt_v, sum_v)
            ones_i = jnp.ones((L,), jnp.int32)

            def body_b(i, carry):
                zs, zc, nv = carry
                l16 = row_v[pl.ds(i * L, L)]
                x = l16 * rinv - mx
                e = jnp.exp(x)
                row_v[pl.ds(i * L, L)] = e
                valid = e >= mp_sc
                ev = jnp.where(valid, e, 0.0)
                y = ev - zc
                t = zs + y
                zc = (t - zs) - y
                zs = t
                nv = nv + valid.astype(jnp.int32)
                bits = lax.bitcast_convert_type(e, jnp.int32)
                b1i = lax.shift_right_logical(bits, 21)
                plsc.addupdate_scatter(cnt_v, [b1i], ones_i, mask=valid)
                plsc.addupdate_scatter(sum_v, [b1i], e, mask=valid)
                return zs, zc, nv

            zs, _, nv = lax.fori_loop(
                0, NVREG, body_b,
                (_splat_f(0.0), _splat_f(0.0), _splat_i(0)))
            z1 = jnp.sum(zs)
            n = jnp.sum(nv)
            pos_k = n - k_sc
            tz1 = (jnp.float32(1.0) - p_sc) * z1

            b1, cbel1, c1, sb1, _ = _descend(
                cnt_v, sum_v, 512, pos_k, tz1, jnp.float32(0.0))
            jrem1 = pos_k - cbel1
            tp_fallback = c1 >= 512

            _zero_hists(cnt_v, sum_v)

            def body_c(i, _):
                e = row_v[pl.ds(i * L, L)]
                valid = e >= mp_sc
                bits = lax.bitcast_convert_type(e, jnp.int32)
                hi = lax.shift_right_logical(bits, 21)
                mid = jnp.bitwise_and(
                    lax.shift_right_logical(bits, 10), 2047)
                mtk = jnp.logical_and(valid, hi == b1)
                mtp = jnp.logical_and(valid, hi == c1)
                plsc.addupdate_scatter(cnt_v, [mid], ones_i, mask=mtk)
                plsc.addupdate_scatter(sum_v, [mid], e, mask=mtp)
                return 0

            lax.fori_loop(0, NVREG, body_c, 0)
            b2, cbel2, c2, sb2, lne2 = _descend(
                cnt_v, sum_v, 2048, jrem1, tz1, sb1)
            jrem2 = jrem1 - cbel2
            c2 = jnp.minimum(c2, jnp.maximum(lne2, 0))

            _zero_hists(cnt_v, sum_v)

            def body_d(i, _):
                e = row_v[pl.ds(i * L, L)]
                valid = e >= mp_sc
                bits = lax.bitcast_convert_type(e, jnp.int32)
                hi = lax.shift_right_logical(bits, 21)
                mid = jnp.bitwise_and(
                    lax.shift_right_logical(bits, 10), 2047)
                lo = jnp.bitwise_and(bits, 1023)
                mtk = jnp.logical_and(
                    jnp.logical_and(valid, hi == b1), mid == b2)
                mtp = jnp.logical_and(
                    jnp.logical_and(valid, hi == c1), mid == c2)
                plsc.addupdate_scatter(cnt_v, [lo], ones_i, mask=mtk)
                plsc.addupdate_scatter(sum_v, [lo], e, mask=mtp)
                return 0

            lax.fori_loop(0, NVREG, body_d, 0)
            b3, _, c3, _, lne3 = _descend(
                cnt_v, sum_v, 1024, jrem2, tz1, sb2)
            c3 = jnp.minimum(c3, jnp.maximum(lne3, 0))

            tk_bits = _splat_i(
                jnp.bitwise_or(
                    jnp.bitwise_or(lax.shift_left(b1, 21),
                                   lax.shift_left(b2, 10)), b3))
            tp_bits = _splat_i(
                jnp.bitwise_or(
                    jnp.bitwise_or(lax.shift_left(c1, 21),
                                   lax.shift_left(c2, 10)), c3))
            tk_e = lax.bitcast_convert_type(tk_bits, jnp.float32)
            tk_e = jnp.where(_splat_i(n) >= k_sc, tk_e, _splat_f(0.0))
            tp_e = jnp.where(tp_fallback, _splat_f(1.0),
                             lax.bitcast_convert_type(tp_bits, jnp.float32))

            cps = [None, None]
            cps[0] = pltpu.async_copy(
                qr_hbm.at[pl.ds(r * V, CH)], qrbufs[0], sems[0])
            wval = _splat_f(-1.0)
            widx = _splat_i(0)
            for c in range(NCH):
                nb = (c + 1) % 2
                if c + 1 < NCH:
                    cps[nb] = pltpu.async_copy(
                        qr_hbm.at[pl.ds(r * V + (c + 1) * CH, CH)],
                        qrbufs[nb], sems[nb])
                cps[c % 2].wait()

                qb = qrbufs[c % 2]

                def body_e(i, carry, _c=c, _qb=qb):
                    wv, wi = carry
                    e = row_v[pl.ds(_c * CH + i * L, L)]
                    q = _qb[pl.ds(i * L, L)]
                    valid = e >= mp_sc
                    kept = jnp.logical_and(
                        jnp.logical_and(valid, e >= tk_e), e >= tp_e)
                    v = jnp.where(kept, e * q, -1.0)
                    gidx = _splat_i(_c * CH + i * L) + it
                    take = v > wv
                    return jnp.where(take, v, wv), jnp.where(take, gidx, wi)

                wval, widx = lax.fori_loop(0, CHV, body_e, (wval, widx))

            wmax = jnp.max(wval)
            winner = jnp.min(jnp.where(wval == wmax, widx, BIG_I32))
            res = jnp.where(t_sc < _EPS, greedy, winner)
            res_v[...] = _splat_i(res)
            pltpu.sync_copy(res_v, out_hbm.at[pl.ds(r * L, L)])

    return k(logits, temperature, top_k, top_p, min_p, qrecip)


def kernel(logits, temperature, top_k, top_p, min_p, q_uniform):
    logits = logits.astype(jnp.float32)
    qrecip = 1.0 / (-jnp.log(jnp.clip(q_uniform, 1e-10, 1.0)))
    out = _sc_sampler(logits.reshape(-1), temperature,
                      top_k.astype(jnp.int32), top_p, min_p,
                      qrecip.reshape(-1))
    return out.reshape(B, L)[:, :1]


# speedup vs baseline: 11.2775x; 1.1306x over previous
import functools

import jax
import jax.numpy as jnp
from jax import lax
from jax.experimental import pallas as pl
from jax.experimental.pallas import tpu as pltpu
from jax.experimental.pallas import tpu_sc as plsc

_EPS = 1e-05
B, V = 64, 100000
L = 16
NVREG = V // L
CH = 10000
NCH = V // CH
CHV = CH // L
NROWS_PER_W = 2
BIG_I32 = 2**30
UNROLL = 5


def _iota16():
    return lax.broadcasted_iota(jnp.int32, (L,), 0)


def _splat_f(x):
    return jnp.full((L,), x, jnp.float32)


def _splat_i(x):
    return jnp.full((L,), x, jnp.int32)


def _row_scalar(vref, r):
    return vref[pl.ds(r, L)][0]


def _descend(cnt_ref, sum_ref, nbuckets, pos_k, tz1, s_base):
    ngroups = nbuckets // L
    it = _iota16()

    def body(g, carry):
        crun, brun, cbel, srun, selrun, sbel, lne = carry
        c16 = cnt_ref[pl.ds(g * L, L)]
        s16 = sum_ref[pl.ds(g * L, L)]
        cinc = plsc.cumsum(c16) + crun
        sinc = plsc.cumsum(s16) + srun
        mle = cinc <= pos_k
        msl = sinc <= tz1
        brun = brun + plsc.all_reduce_population_count(mle)
        selrun = selrun + plsc.all_reduce_population_count(msl)
        cbel = jnp.maximum(cbel, jnp.where(mle, cinc, 0))
        sbel = jnp.maximum(sbel, jnp.where(msl, sinc, s_base))
        gidx = _splat_i(g * L) + it
        lne = jnp.maximum(lne, jnp.where(s16 > 0.0, gidx, -1))
        crun = jnp.max(cinc)
        srun = jnp.max(sinc)
        return crun, brun, cbel, srun, selrun, sbel, lne

    init = (jnp.int32(0), _splat_i(0), _splat_i(0), jnp.float32(s_base),
            _splat_i(0), _splat_f(s_base), _splat_i(-1))
    _, brun, cbel, _, selrun, sbel, lne = lax.fori_loop(
        0, ngroups, body, init)
    return (jnp.max(brun), jnp.max(cbel), jnp.max(selrun), jnp.max(sbel),
            jnp.max(lne))


def _zero_hists(cnt_ref, sum_ref):
    zi = jnp.zeros((L,), jnp.int32)
    zf = jnp.zeros((L,), jnp.float32)

    def zbody(z, _):
        cnt_ref[pl.ds(z * L, L)] = zi
        sum_ref[pl.ds(z * L, L)] = zf
        return 0

    lax.fori_loop(0, 2048 // L, zbody, 0)


def _sc_sampler(logits, temperature, top_k, top_p, min_p, qrecip):
    mesh = plsc.VectorSubcoreMesh(core_axis_name="c", subcore_axis_name="s")

    @functools.partial(
        pl.kernel,
        out_type=jax.ShapeDtypeStruct((B * L,), jnp.int32),
        mesh=mesh,
        compiler_params=pltpu.CompilerParams(needs_layout_passes=False),
        scratch_types=[
            pltpu.VMEM((V,), jnp.float32),
            pltpu.VMEM((CH,), jnp.float32),
            pltpu.VMEM((CH,), jnp.float32),
            pltpu.VMEM((2048,), jnp.int32),
            pltpu.VMEM((2048,), jnp.float32),
            pltpu.VMEM((B + L,), jnp.float32),
            pltpu.VMEM((B + L,), jnp.int32),
            pltpu.VMEM((B + L,), jnp.float32),
            pltpu.VMEM((B + L,), jnp.float32),
            pltpu.VMEM((L,), jnp.int32),
            pltpu.SemaphoreType.DMA,
            pltpu.SemaphoreType.DMA,
        ],
    )
    def k(logits_hbm, temp_hbm, topk_hbm, topp_hbm, minp_hbm, qr_hbm,
          out_hbm, row_v, qr0_v, qr1_v, cnt_v, sum_v, t_v, k_v, p_v,
          mp_v, res_v, sem0, sem1):
        qrbufs = (qr0_v, qr1_v)
        wid = lax.axis_index("s") * 2 + lax.axis_index("c")
        pltpu.sync_copy(temp_hbm, t_v.at[pl.ds(0, B)])
        pltpu.sync_copy(topk_hbm, k_v.at[pl.ds(0, B)])
        pltpu.sync_copy(topp_hbm, p_v.at[pl.ds(0, B)])
        pltpu.sync_copy(minp_hbm, mp_v.at[pl.ds(0, B)])
        it = _iota16()
        sems = (sem0, sem1)

        for j in range(NROWS_PER_W):
            r = wid * NROWS_PER_W + j
            pltpu.sync_copy(logits_hbm.at[pl.ds(r * V, V)], row_v)
            t_sc = _row_scalar(t_v, r)
            k_sc = _row_scalar(k_v, r)
            p_sc = _row_scalar(p_v, r)
            mp_sc = _row_scalar(mp_v, r)
            t16 = _splat_f(t_sc)
            t_eff = jnp.where(t16 < _EPS, _splat_f(1.0), t16)
            rinv = _splat_f(1.0) / t_eff

            def body_a(i, carry):
                out = []
                for u in range(UNROLL):
                    bmax, bidx = carry[2 * u], carry[2 * u + 1]
                    base = (i * UNROLL + u) * L
                    v = row_v[pl.ds(base, L)]
                    gidx = _splat_i(base) + it
                    take = v > bmax
                    out.append(jnp.where(take, v, bmax))
                    out.append(jnp.where(take, gidx, bidx))
                return tuple(out)

            init_a = (_splat_f(-jnp.inf), _splat_i(0)) * UNROLL
            acc_a = lax.fori_loop(0, NVREG // UNROLL, body_a, init_a)
            bmax, bidx = acc_a[0], acc_a[1]
            for u in range(1, UNROLL):
                v, gi = acc_a[2 * u], acc_a[2 * u + 1]
                take = jnp.logical_or(
                    v > bmax, jnp.logical_and(v == bmax, gi < bidx))
                bmax = jnp.where(take, v, bmax)
                bidx = jnp.where(take, gi, bidx)
            m_raw = jnp.max(bmax)
            greedy = jnp.min(jnp.where(bmax == m_raw, bidx, BIG_I32))

            mx = _splat_f(m_raw) * rinv

            _zero_hists(cnt_v, sum_v)
            ones_i = jnp.ones((L,), jnp.int32)

            def body_b(i, carry):
                out = []
                for u in range(UNROLL):
                    zs, zc, nv = (carry[3 * u], carry[3 * u + 1],
                                  carry[3 * u + 2])
                    base = (i * UNROLL + u) * L
                    l16 = row_v[pl.ds(base, L)]
                    x = l16 * rinv - mx
                    e = jnp.exp(x)
                    row_v[pl.ds(base, L)] = e
                    valid = e >= mp_sc
                    ev = jnp.where(valid, e, 0.0)
                    y = ev - zc
                    t = zs + y
                    zc = (t - zs) - y
                    zs = t
                    nv = nv + valid.astype(jnp.int32)
                    bits = lax.bitcast_convert_type(e, jnp.int32)
                    b1i = lax.shift_right_logical(bits, 21)
                    plsc.addupdate_scatter(cnt_v, [b1i], ones_i, mask=valid)
                    plsc.addupdate_scatter(sum_v, [b1i], e, mask=valid)
                    out.extend((zs, zc, nv))
                return tuple(out)

            init_b = (_splat_f(0.0), _splat_f(0.0), _splat_i(0)) * UNROLL
            acc_b = lax.fori_loop(0, NVREG // UNROLL, body_b, init_b)
            z1 = jnp.float32(0.0)
            n = jnp.int32(0)
            for u in range(UNROLL):
                z1 = z1 + jnp.sum(acc_b[3 * u])
                n = n + jnp.sum(acc_b[3 * u + 2])
            pos_k = n - k_sc
            tz1 = (jnp.float32(1.0) - p_sc) * z1

            b1, cbel1, c1, sb1, _ = _descend(
                cnt_v, sum_v, 512, pos_k, tz1, jnp.float32(0.0))
            jrem1 = pos_k - cbel1
            tp_fallback = c1 >= 512

            _zero_hists(cnt_v, sum_v)

            def body_c(i, _):
                for u in range(UNROLL):
                    base = (i * UNROLL + u) * L
                    e = row_v[pl.ds(base, L)]
                    valid = e >= mp_sc
                    bits = lax.bitcast_convert_type(e, jnp.int32)
                    hi = lax.shift_right_logical(bits, 21)
                    mid = jnp.bitwise_and(
                        lax.shift_right_logical(bits, 10), 2047)
                    mtk = jnp.logical_and(valid, hi == b1)
                    mtp = jnp.logical_and(valid, hi == c1)
                    plsc.addupdate_scatter(cnt_v, [mid], ones_i, mask=mtk)
                    plsc.addupdate_scatter(sum_v, [mid], e, mask=mtp)
                return 0

            lax.fori_loop(0, NVREG // UNROLL, body_c, 0)
            b2, cbel2, c2, sb2, lne2 = _descend(
                cnt_v, sum_v, 2048, jrem1, tz1, sb1)
            jrem2 = jrem1 - cbel2
            c2 = jnp.minimum(c2, jnp.maximum(lne2, 0))

            _zero_hists(cnt_v, sum_v)

            def body_d(i, _):
                for u in range(UNROLL):
                    base = (i * UNROLL + u) * L
                    e = row_v[pl.ds(base, L)]
                    valid = e >= mp_sc
                    bits = lax.bitcast_convert_type(e, jnp.int32)
                    hi = lax.shift_right_logical(bits, 21)
                    mid = jnp.bitwise_and(
                        lax.shift_right_logical(bits, 10), 2047)
                    lo = jnp.bitwise_and(bits, 1023)
                    mtk = jnp.logical_and(
                        jnp.logical_and(valid, hi == b1), mid == b2)
                    mtp = jnp.logical_and(
                        jnp.logical_and(valid, hi == c1), mid == c2)
                    plsc.addupdate_scatter(cnt_v, [lo], ones_i, mask=mtk)
                    plsc.addupdate_scatter(sum_v, [lo], e, mask=mtp)
                return 0

            lax.fori_loop(0, NVREG // UNROLL, body_d, 0)
            b3, _, c3, _, lne3 = _descend(
                cnt_v, sum_v, 1024, jrem2, tz1, sb2)
            c3 = jnp.minimum(c3, jnp.maximum(lne3, 0))

            tk_bits = _splat_i(
                jnp.bitwise_or(
                    jnp.bitwise_or(lax.shift_left(b1, 21),
                                   lax.shift_left(b2, 10)), b3))
            tp_bits = _splat_i(
                jnp.bitwise_or(
                    jnp.bitwise_or(lax.shift_left(c1, 21),
                                   lax.shift_left(c2, 10)), c3))
            tk_e = lax.bitcast_convert_type(tk_bits, jnp.float32)
            tk_e = jnp.where(_splat_i(n) >= k_sc, tk_e, _splat_f(0.0))
            tp_e = jnp.where(tp_fallback, _splat_f(1.0),
                             lax.bitcast_convert_type(tp_bits, jnp.float32))

            cps = [None, None]
            cps[0] = pltpu.async_copy(
                qr_hbm.at[pl.ds(r * V, CH)], qrbufs[0], sems[0])
            acc_e = (_splat_f(-1.0), _splat_i(0)) * UNROLL
            for c in range(NCH):
                nb = (c + 1) % 2
                if c + 1 < NCH:
                    cps[nb] = pltpu.async_copy(
                        qr_hbm.at[pl.ds(r * V + (c + 1) * CH, CH)],
                        qrbufs[nb], sems[nb])
                cps[c % 2].wait()

                qb = qrbufs[c % 2]

                def body_e(i, carry, _c=c, _qb=qb):
                    out = []
                    for u in range(UNROLL):
                        wv, wi = carry[2 * u], carry[2 * u + 1]
                        off = (i * UNROLL + u) * L
                        e = row_v[pl.ds(_c * CH + off, L)]
                        q = _qb[pl.ds(off, L)]
                        valid = e >= mp_sc
                        kept = jnp.logical_and(
                            jnp.logical_and(valid, e >= tk_e), e >= tp_e)
                        v = jnp.where(kept, e * q, -1.0)
                        gidx = _splat_i(_c * CH + off) + it
                        take = v > wv
                        out.append(jnp.where(take, v, wv))
                        out.append(jnp.where(take, gidx, wi))
                    return tuple(out)

                acc_e = lax.fori_loop(0, CHV // UNROLL, body_e, acc_e)

            wval, widx = acc_e[0], acc_e[1]
            for u in range(1, UNROLL):
                v, gi = acc_e[2 * u], acc_e[2 * u + 1]
                take = jnp.logical_or(
                    v > wval, jnp.logical_and(v == wval, gi < widx))
                wval = jnp.where(take, v, wval)
                widx = jnp.where(take, gi, widx)
            wmax = jnp.max(wval)
            winner = jnp.min(jnp.where(wval == wmax, widx, BIG_I32))
            res = jnp.where(t_sc < _EPS, greedy, winner)
            res_v[...] = _splat_i(res)
            pltpu.sync_copy(res_v, out_hbm.at[pl.ds(r * L, L)])

    return k(logits, temperature, top_k, top_p, min_p, qrecip)


def kernel(logits, temperature, top_k, top_p, min_p, q_uniform):
    logits = logits.astype(jnp.float32)
    qrecip = 1.0 / (-jnp.log(jnp.clip(q_uniform, 1e-10, 1.0)))
    out = _sc_sampler(logits.reshape(-1), temperature,
                      top_k.astype(jnp.int32), top_p, min_p,
                      qrecip.reshape(-1))
    return out.reshape(B, L)[:, :1]


# speedup vs baseline: 26.4708x; 2.3472x over previous
import functools

import jax
import jax.numpy as jnp
from jax import lax
from jax.experimental import pallas as pl
from jax.experimental.pallas import tpu as pltpu
from jax.experimental.pallas import tpu_sc as plsc

_EPS = 1e-05
B, V = 64, 100000
L = 16
NVREG = V // L
CH = 10000
NCH = V // CH
CHV = CH // L
NROWS_PER_W = 2
BIG_I32 = 2**30
UNROLL = 5


def _iota16():
    return lax.broadcasted_iota(jnp.int32, (L,), 0)


def _splat_f(x):
    return jnp.full((L,), x, jnp.float32)


def _splat_i(x):
    return jnp.full((L,), x, jnp.int32)


def _row_scalar(vref, r):
    return vref[pl.ds(r, L)][0]


def _descend(cnt_ref, sum_ref, nbuckets, pos_k, tz1, s_base):
    ngroups = nbuckets // L
    it = _iota16()

    def body(g, carry):
        crun, brun, cbel, srun, selrun, sbel, lne = carry
        c16 = cnt_ref[pl.ds(g * L, L)]
        s16 = sum_ref[pl.ds(g * L, L)]
        cinc = plsc.cumsum(c16) + crun
        sinc = plsc.cumsum(s16) + srun
        mle = cinc <= pos_k
        msl = sinc <= tz1
        brun = brun + plsc.all_reduce_population_count(mle)
        selrun = selrun + plsc.all_reduce_population_count(msl)
        cbel = jnp.maximum(cbel, jnp.where(mle, cinc, 0))
        sbel = jnp.maximum(sbel, jnp.where(msl, sinc, s_base))
        gidx = _splat_i(g * L) + it
        lne = jnp.maximum(lne, jnp.where(s16 > 0.0, gidx, -1))
        crun = jnp.max(cinc)
        srun = jnp.max(sinc)
        return crun, brun, cbel, srun, selrun, sbel, lne

    init = (jnp.int32(0), _splat_i(0), _splat_i(0), jnp.float32(s_base),
            _splat_i(0), _splat_f(s_base), _splat_i(-1))
    _, brun, cbel, _, selrun, sbel, lne = lax.fori_loop(
        0, ngroups, body, init)
    return (jnp.max(brun), jnp.max(cbel), jnp.max(selrun), jnp.max(sbel),
            jnp.max(lne))


def _zero_hists(cnt_ref, sum_ref):
    zi = jnp.zeros((L,), jnp.int32)
    zf = jnp.zeros((L,), jnp.float32)

    def zbody(z, _):
        cnt_ref[pl.ds(z * L, L)] = zi
        sum_ref[pl.ds(z * L, L)] = zf
        return 0

    lax.fori_loop(0, 2048 // L, zbody, 0)


def _sc_sampler(logits, temperature, top_k, top_p, min_p, qrecip):
    mesh = plsc.VectorSubcoreMesh(core_axis_name="c", subcore_axis_name="s")

    @functools.partial(
        pl.kernel,
        out_type=jax.ShapeDtypeStruct((B * L,), jnp.int32),
        mesh=mesh,
        compiler_params=pltpu.CompilerParams(needs_layout_passes=False),
        scratch_types=[
            pltpu.VMEM((V,), jnp.float32),
            pltpu.VMEM((CH,), jnp.float32),
            pltpu.VMEM((CH,), jnp.float32),
            pltpu.VMEM((2048,), jnp.int32),
            pltpu.VMEM((2048,), jnp.float32),
            pltpu.VMEM((B + L,), jnp.float32),
            pltpu.VMEM((B + L,), jnp.int32),
            pltpu.VMEM((B + L,), jnp.float32),
            pltpu.VMEM((B + L,), jnp.float32),
            pltpu.VMEM((L,), jnp.int32),
            pltpu.SemaphoreType.DMA,
            pltpu.SemaphoreType.DMA,
        ],
    )
    def k(logits_hbm, temp_hbm, topk_hbm, topp_hbm, minp_hbm, qr_hbm,
          out_hbm, row_v, qr0_v, qr1_v, cnt_v, sum_v, t_v, k_v, p_v,
          mp_v, res_v, sem0, sem1):
        qrbufs = (qr0_v, qr1_v)
        wid = lax.axis_index("s") * 2 + lax.axis_index("c")
        pltpu.sync_copy(temp_hbm, t_v.at[pl.ds(0, B)])
        pltpu.sync_copy(topk_hbm, k_v.at[pl.ds(0, B)])
        pltpu.sync_copy(topp_hbm, p_v.at[pl.ds(0, B)])
        pltpu.sync_copy(minp_hbm, mp_v.at[pl.ds(0, B)])
        it = _iota16()
        sems = (sem0, sem1)

        for j in range(NROWS_PER_W):
            r = wid * NROWS_PER_W + j
            pltpu.sync_copy(logits_hbm.at[pl.ds(r * V, V)], row_v)
            t_sc = _row_scalar(t_v, r)
            k_sc = _row_scalar(k_v, r)
            p_sc = _row_scalar(p_v, r)
            mp_sc = _row_scalar(mp_v, r)
            t16 = _splat_f(t_sc)
            t_eff = jnp.where(t16 < _EPS, _splat_f(1.0), t16)
            rinv = _splat_f(1.0) / t_eff

            def body_a(i, carry):
                out = []
                for u in range(UNROLL):
                    bmax, bidx = carry[2 * u], carry[2 * u + 1]
                    base = (i * UNROLL + u) * L
                    v = row_v[pl.ds(base, L)]
                    gidx = _splat_i(base) + it
                    take = v > bmax
                    out.append(jnp.where(take, v, bmax))
                    out.append(jnp.where(take, gidx, bidx))
                return tuple(out)

            init_a = (_splat_f(-jnp.inf), _splat_i(0)) * UNROLL
            acc_a = plsc.parallel_loop(
                0, NVREG // UNROLL, carry=init_a)(
                    lambda i, c: body_a(i, c))
            bmax, bidx = acc_a[0], acc_a[1]
            for u in range(1, UNROLL):
                v, gi = acc_a[2 * u], acc_a[2 * u + 1]
                take = jnp.logical_or(
                    v > bmax, jnp.logical_and(v == bmax, gi < bidx))
                bmax = jnp.where(take, v, bmax)
                bidx = jnp.where(take, gi, bidx)
            m_raw = jnp.max(bmax)
            greedy = jnp.min(jnp.where(bmax == m_raw, bidx, BIG_I32))

            mx = _splat_f(m_raw) * rinv

            _zero_hists(cnt_v, sum_v)
            ones_i = jnp.ones((L,), jnp.int32)

            def body_b(i, carry):
                out = []
                for u in range(UNROLL):
                    zs, zc, nv = (carry[3 * u], carry[3 * u + 1],
                                  carry[3 * u + 2])
                    base = (i * UNROLL + u) * L
                    l16 = row_v[pl.ds(base, L)]
                    x = l16 * rinv - mx
                    e = jnp.exp(x)
                    row_v[pl.ds(base, L)] = e
                    valid = e >= mp_sc
                    ev = jnp.where(valid, e, 0.0)
                    y = ev - zc
                    t = zs + y
                    zc = (t - zs) - y
                    zs = t
                    nv = nv + valid.astype(jnp.int32)
                    bits = lax.bitcast_convert_type(e, jnp.int32)
                    b1i = lax.shift_right_logical(bits, 21)
                    plsc.addupdate_scatter(cnt_v, [b1i], ones_i, mask=valid)
                    plsc.addupdate_scatter(sum_v, [b1i], e, mask=valid)
                    out.extend((zs, zc, nv))
                return tuple(out)

            init_b = (_splat_f(0.0), _splat_f(0.0), _splat_i(0)) * UNROLL
            acc_b = plsc.parallel_loop(
                0, NVREG // UNROLL, carry=init_b)(
                    lambda i, c: body_b(i, c))
            z1 = jnp.float32(0.0)
            n = jnp.int32(0)
            for u in range(UNROLL):
                z1 = z1 + jnp.sum(acc_b[3 * u])
                n = n + jnp.sum(acc_b[3 * u + 2])
            pos_k = n - k_sc
            tz1 = (jnp.float32(1.0) - p_sc) * z1

            b1, cbel1, c1, sb1, _ = _descend(
                cnt_v, sum_v, 512, pos_k, tz1, jnp.float32(0.0))
            jrem1 = pos_k - cbel1
            tp_fallback = c1 >= 512

            _zero_hists(cnt_v, sum_v)

            def body_c(i):
                for u in range(UNROLL):
                    base = (i * UNROLL + u) * L
                    e = row_v[pl.ds(base, L)]
                    valid = e >= mp_sc
                    bits = lax.bitcast_convert_type(e, jnp.int32)
                    hi = lax.shift_right_logical(bits, 21)
                    mid = jnp.bitwise_and(
                        lax.shift_right_logical(bits, 10), 2047)
                    mtk = jnp.logical_and(valid, hi == b1)
                    mtp = jnp.logical_and(valid, hi == c1)
                    plsc.addupdate_scatter(cnt_v, [mid], ones_i, mask=mtk)
                    plsc.addupdate_scatter(sum_v, [mid], e, mask=mtp)

            plsc.parallel_loop(0, NVREG // UNROLL)(body_c)
            b2, cbel2, c2, sb2, lne2 = _descend(
                cnt_v, sum_v, 2048, jrem1, tz1, sb1)
            jrem2 = jrem1 - cbel2
            c2 = jnp.minimum(c2, jnp.maximum(lne2, 0))

            _zero_hists(cnt_v, sum_v)

            def body_d(i):
                for u in range(UNROLL):
                    base = (i * UNROLL + u) * L
                    e = row_v[pl.ds(base, L)]
                    valid = e >= mp_sc
                    bits = lax.bitcast_convert_type(e, jnp.int32)
                    hi = lax.shift_right_logical(bits, 21)
                    mid = jnp.bitwise_and(
                        lax.shift_right_logical(bits, 10), 2047)
                    lo = jnp.bitwise_and(bits, 1023)
                    mtk = jnp.logical_and(
                        jnp.logical_and(valid, hi == b1), mid == b2)
                    mtp = jnp.logical_and(
                        jnp.logical_and(valid, hi == c1), mid == c2)
                    plsc.addupdate_scatter(cnt_v, [lo], ones_i, mask=mtk)
                    plsc.addupdate_scatter(sum_v, [lo], e, mask=mtp)

            plsc.parallel_loop(0, NVREG // UNROLL)(body_d)
            b3, _, c3, _, lne3 = _descend(
                cnt_v, sum_v, 1024, jrem2, tz1, sb2)
            c3 = jnp.minimum(c3, jnp.maximum(lne3, 0))

            tk_bits = _splat_i(
                jnp.bitwise_or(
                    jnp.bitwise_or(lax.shift_left(b1, 21),
                                   lax.shift_left(b2, 10)), b3))
            tp_bits = _splat_i(
                jnp.bitwise_or(
                    jnp.bitwise_or(lax.shift_left(c1, 21),
                                   lax.shift_left(c2, 10)), c3))
            tk_e = lax.bitcast_convert_type(tk_bits, jnp.float32)
            tk_e = jnp.where(_splat_i(n) >= k_sc, tk_e, _splat_f(0.0))
            tp_e = jnp.where(tp_fallback, _splat_f(1.0),
                             lax.bitcast_convert_type(tp_bits, jnp.float32))

            cps = [None, None]
            cps[0] = pltpu.async_copy(
                qr_hbm.at[pl.ds(r * V, CH)], qrbufs[0], sems[0])
            acc_e = (_splat_f(-1.0), _splat_i(0)) * UNROLL
            for c in range(NCH):
                nb = (c + 1) % 2
                if c + 1 < NCH:
                    cps[nb] = pltpu.async_copy(
                        qr_hbm.at[pl.ds(r * V + (c + 1) * CH, CH)],
                        qrbufs[nb], sems[nb])
                cps[c % 2].wait()

                qb = qrbufs[c % 2]

                def body_e(i, carry, _c=c, _qb=qb):
                    out = []
                    for u in range(UNROLL):
                        wv, wi = carry[2 * u], carry[2 * u + 1]
                        off = (i * UNROLL + u) * L
                        e = row_v[pl.ds(_c * CH + off, L)]
                        q = _qb[pl.ds(off, L)]
                        valid = e >= mp_sc
                        kept = jnp.logical_and(
                            jnp.logical_and(valid, e >= tk_e), e >= tp_e)
                        v = jnp.where(kept, e * q, -1.0)
                        gidx = _splat_i(_c * CH + off) + it
                        take = v > wv
                        out.append(jnp.where(take, v, wv))
                        out.append(jnp.where(take, gidx, wi))
                    return tuple(out)

                acc_e = plsc.parallel_loop(
                    0, CHV // UNROLL, carry=acc_e)(
                        lambda i, cc, _f=body_e: _f(i, cc))

            wval, widx = acc_e[0], acc_e[1]
            for u in range(1, UNROLL):
                v, gi = acc_e[2 * u], acc_e[2 * u + 1]
                take = jnp.logical_or(
                    v > wval, jnp.logical_and(v == wval, gi < widx))
                wval = jnp.where(take, v, wval)
                widx = jnp.where(take, gi, widx)
            wmax = jnp.max(wval)
            winner = jnp.min(jnp.where(wval == wmax, widx, BIG_I32))
            res = jnp.where(t_sc < _EPS, greedy, winner)
            res_v[...] = _splat_i(res)
            pltpu.sync_copy(res_v, out_hbm.at[pl.ds(r * L, L)])

    return k(logits, temperature, top_k, top_p, min_p, qrecip)


def kernel(logits, temperature, top_k, top_p, min_p, q_uniform):
    logits = logits.astype(jnp.float32)
    qrecip = 1.0 / (-jnp.log(jnp.clip(q_uniform, 1e-10, 1.0)))
    out = _sc_sampler(logits.reshape(-1), temperature,
                      top_k.astype(jnp.int32), top_p, min_p,
                      qrecip.reshape(-1))
    return out.reshape(B, L)[:, :1]
